# Initial kernel scaffold; baseline (speedup 1.0000x reference)
import jax
import jax.numpy as jnp
from jax.experimental import pallas as pl


def kernel(x, adj, W0, W1, W2, W3, W4, b0, b1, b2, b3, b4):
    raise NotImplementedError("write your pallas kernel here")


# speedup vs baseline: 9.0732x; 9.0732x over previous
import functools

import jax
import jax.numpy as jnp
from jax import lax
from jax.experimental import pallas as pl
from jax.experimental.pallas import tpu as pltpu
from jax.experimental.pallas import tpu_sc as plsc

_N = 10000
_D = 128
_L = 5
_NC = 2
_NS = 16
_NPAD = 10240
_TR = _NPAD // _NS
_CH = 128
_NCH = 79
_EPAD = _NC * _NS * _NCH * _CH
_BN = 512

_sc_mesh = plsc.VectorSubcoreMesh(core_axis_name="c", subcore_axis_name="s")


@functools.partial(
    pl.kernel,
    out_type=jax.ShapeDtypeStruct((_NC * _NPAD, 16), jnp.float32),
    mesh=_sc_mesh,
    scratch_types=[
        pltpu.VMEM((_NCH, _CH), jnp.int32),
        pltpu.VMEM((_CH, 16), jnp.float32),
        pltpu.VMEM_SHARED((_NPAD, 16), jnp.float32),
    ],
)
def _sc_deg(dst_hbm, ones_hbm, z16_hbm, out_hbm, idx_v, ones_v, acc):
    c = lax.axis_index("c")
    s = lax.axis_index("s")
    w = c * _NS + s
    pltpu.sync_copy(z16_hbm, acc.at[pl.ds(s * _TR, _TR)])
    pltpu.sync_copy(ones_hbm, ones_v)
    pltpu.sync_copy(dst_hbm.at[w], idx_v)
    plsc.subcore_barrier()

    def step(j, carry):
        pltpu.sync_copy(ones_v, acc.at[idx_v.at[j]], add=True)
        return carry

    lax.fori_loop(0, _NCH, step, 0)
    plsc.subcore_barrier()
    pltpu.sync_copy(acc.at[pl.ds(s * _TR, _TR)],
                    out_hbm.at[pl.ds(c * _NPAD + s * _TR, _TR)])


@functools.partial(
    pl.kernel,
    out_type=jax.ShapeDtypeStruct((_NC * _NPAD, _D), jnp.float32),
    mesh=_sc_mesh,
    scratch_types=[
        pltpu.VMEM((_NCH, _CH), jnp.int32),
        pltpu.VMEM((_NCH, _CH), jnp.int32),
        pltpu.VMEM((_CH, _D), jnp.float32),
        pltpu.VMEM_SHARED((_NPAD, _D), jnp.float32),
        pltpu.SemaphoreType.DMA,
    ],
)
def _sc_agg(g_hbm, src_hbm, dst_hbm, zrows_hbm, out_hbm,
            src_v, dst_v, rows, acc, sem):
    c = lax.axis_index("c")
    s = lax.axis_index("s")
    w = c * _NS + s
    pltpu.sync_copy(zrows_hbm, acc.at[pl.ds(s * _TR, _TR)])
    pltpu.sync_copy(src_hbm.at[w], src_v)
    pltpu.sync_copy(dst_hbm.at[w], dst_v)
    plsc.subcore_barrier()

    def step(j, carry):
        pltpu.async_copy(g_hbm.at[src_v.at[j]], rows, sem).wait()
        pltpu.sync_copy(rows, acc.at[dst_v.at[j]], add=True)
        return carry

    lax.fori_loop(0, _NCH, step, 0)
    plsc.subcore_barrier()
    pltpu.sync_copy(acc.at[pl.ds(s * _TR, _TR)],
                    out_hbm.at[pl.ds(c * _NPAD + s * _TR, _TR)])


def _dinv_of(dg):
    return lax.rsqrt(dg[0, :, 0:1] + dg[1, :, 0:1] + 1.0)


def _tc_first_body(dg_ref, x_ref, w_ref, o_ref):
    dinv = _dinv_of(dg_ref[...])
    h = jnp.dot(x_ref[...], w_ref[...], preferred_element_type=jnp.float32)
    o_ref[...] = dinv * h


def _tc_layer_body(dg_ref, p_ref, g_ref, b_ref, w_ref, o_ref):
    dinv = _dinv_of(dg_ref[...])
    pv = p_ref[...]
    t = dinv * (pv[0] + pv[1] + g_ref[...]) + b_ref[...]
    t = jnp.maximum(t, 0.0)
    o_ref[...] = dinv * jnp.dot(t, w_ref[...],
                                preferred_element_type=jnp.float32)


def _tc_final_body(dg_ref, p_ref, g_ref, b_ref, o_ref):
    dinv = _dinv_of(dg_ref[...])
    pv = p_ref[...]
    o_ref[...] = dinv * (pv[0] + pv[1] + g_ref[...]) + b_ref[...]


_GRID = (_NPAD // _BN,)
_dg_spec = pl.BlockSpec((2, _BN, 16), lambda i: (0, i, 0))
_row_spec = pl.BlockSpec((_BN, _D), lambda i: (i, 0))
_p_spec = pl.BlockSpec((2, _BN, _D), lambda i: (0, i, 0))
_w_spec = pl.BlockSpec((_D, _D), lambda i: (0, 0))
_b_spec = pl.BlockSpec((1, _D), lambda i: (0, 0))
_out_sds = jax.ShapeDtypeStruct((_NPAD, _D), jnp.float32)

_tc_first = pl.pallas_call(
    _tc_first_body, grid=_GRID,
    in_specs=[_dg_spec, _row_spec, _w_spec],
    out_specs=_row_spec, out_shape=_out_sds)

_tc_layer = pl.pallas_call(
    _tc_layer_body, grid=_GRID,
    in_specs=[_dg_spec, _p_spec, _row_spec, _b_spec, _w_spec],
    out_specs=_row_spec, out_shape=_out_sds)

_tc_final = pl.pallas_call(
    _tc_final_body, grid=_GRID,
    in_specs=[_dg_spec, _p_spec, _row_spec, _b_spec],
    out_specs=_row_spec, out_shape=_out_sds)


def kernel(x, adj, W0, W1, W2, W3, W4, b0, b1, b2, b3, b4):
    Ws = [W0, W1, W2, W3, W4]
    bs = [b0, b1, b2, b3, b4]

    src, dst = adj[0], adj[1]
    npad_e = _EPAD - src.shape[0]
    padv = jnp.full((npad_e,), _N, jnp.int32)
    srcp = jnp.concatenate([src, padv]).reshape(_NC * _NS, _NCH, _CH)
    dstp = jnp.concatenate([dst, padv]).reshape(_NC * _NS, _NCH, _CH)
    xp = jnp.concatenate(
        [x, jnp.zeros((_NPAD - _N, _D), jnp.float32)], axis=0)

    ones16 = jnp.ones((_CH, 16), jnp.float32)
    z16 = jnp.zeros((_TR, 16), jnp.float32)
    zrows = jnp.zeros((_TR, _D), jnp.float32)

    degp = _sc_deg(dstp, ones16, z16).reshape(_NC, _NPAD, 16)
    g = _tc_first(degp, xp, Ws[0])
    for i in range(1, _L):
        p = _sc_agg(g, srcp, dstp, zrows).reshape(_NC, _NPAD, _D)
        g = _tc_layer(degp, p, g, bs[i - 1].reshape(1, _D), Ws[i])
    p = _sc_agg(g, srcp, dstp, zrows).reshape(_NC, _NPAD, _D)
    out = _tc_final(degp, p, g, bs[_L - 1].reshape(1, _D))
    return out[:_N]
